# Initial kernel scaffold; baseline (speedup 1.0000x reference)
import jax
import jax.numpy as jnp
from jax.experimental import pallas as pl


def kernel(signal, window):
    raise NotImplementedError("write your pallas kernel here")


# speedup vs baseline: 1.0297x; 1.0297x over previous
import functools
import math

import jax
import jax.numpy as jnp
import numpy as np
from jax import lax
from jax.experimental import pallas as pl
from jax.experimental.pallas import tpu as pltpu
from jax.experimental.pallas import tpu_sc as plsc

N_FFT = 1024
WIN_LENGTH = 1024
HOP = 256
SR = 44100

CHUNK = 128
NCHUNK = 65
FPAD = CHUNK * NCHUNK
KPAD = 528
TILE_W = 136
NB_F = 513
TILE_WORDS = NB_F * TILE_W
BATCH_FRAMES = 16
BATCH_PTS = BATCH_FRAMES * KPAD
NBATCH = CHUNK // BATCH_FRAMES


def _arg(values):
    return (jnp.angle(values) / (2.0 * np.pi)) % 1.0


def _stft_frames_major(x, window):
    pad = N_FFT // 2
    xp = jnp.pad(x, (pad, pad), mode='reflect')
    n_frames = 1 + (xp.shape[0] - N_FFT) // HOP
    idx = jnp.arange(n_frames)[:, None] * HOP + jnp.arange(N_FFT)[None, :]
    frames = xp[idx] * window[None, :]
    return jnp.fft.rfft(frames, axis=1)


def _prep_body(f_ref, t_ref, w_ref, idx_ref, wm_ref, *, wf, wt, t_hi,
               nb_f, nb_t, block_rows):
    b = pl.program_id(0)
    f = f_ref[...]
    t = t_ref[...]
    w = w_ref[...]
    fi = jnp.clip(jnp.floor((f - 0.0) / wf).astype(jnp.int32), 0, nb_f - 1)
    ti = jnp.clip(jnp.floor(t / wt).astype(jnp.int32), 0, nb_t - 1)
    inb = (f >= 0.0) & (f <= 0.5) & (t >= 0.0) & (t <= t_hi)
    wm = jnp.where(inb, w, jnp.zeros_like(w))
    m = b * block_rows + lax.broadcasted_iota(jnp.int32, f.shape, 0)
    dt = jnp.clip(ti - m, -1, 4)
    mmod = m - (m // CHUNK) * CHUNK
    idx_ref[...] = fi * TILE_W + mmod + 1 + dt
    wm_ref[...] = wm


def _run_prep(f2, t2, w2, nb_f, nb_t, wf, wt, t_hi):
    block_rows = 128
    grid = FPAD // block_rows
    spec = pl.BlockSpec((block_rows, KPAD), lambda b: (b, 0))
    return pl.pallas_call(
        functools.partial(_prep_body, wf=wf, wt=wt, t_hi=t_hi, nb_f=nb_f,
                          nb_t=nb_t, block_rows=block_rows),
        grid=(grid,),
        in_specs=[spec, spec, spec],
        out_specs=[spec, spec],
        out_shape=[
            jax.ShapeDtypeStruct((FPAD, KPAD), jnp.int32),
            jax.ShapeDtypeStruct((FPAD, KPAD), jnp.float32),
        ],
    )(f2, t2, w2)


def _sc_hist_body(idx_hbm, w_hbm, tiles_hbm, idx_v, w_v, hist_v):
    cid = lax.axis_index("c")
    sid = lax.axis_index("s")
    wid = sid * 2 + cid

    def zero_body(i, carry):
        hist_v[pl.ds(i * 16, 16)] = jnp.zeros((16,), jnp.float32)
        return carry

    def scat_body(i, carry):
        iv = idx_v[pl.ds(i * 16, 16)]
        wv = w_v[pl.ds(i * 16, 16)]
        plsc.addupdate_scatter(hist_v, [iv], wv)
        return carry

    for rep in range(3):
        c = rep * 32 + wid

        @pl.when(c < NCHUNK)
        def _():
            lax.fori_loop(0, TILE_WORDS // 16, zero_body, 0)
            for j in range(NBATCH):
                off = pl.multiple_of(c * (CHUNK * KPAD) + j * BATCH_PTS, 8)
                pltpu.sync_copy(idx_hbm.at[pl.ds(off, BATCH_PTS)], idx_v)
                pltpu.sync_copy(w_hbm.at[pl.ds(off, BATCH_PTS)], w_v)
                lax.fori_loop(0, BATCH_PTS // 16, scat_body, 0)
            pltpu.sync_copy(hist_v, tiles_hbm.at[c])


def _run_sc_hist(idx_flat, w_flat):
    mesh = plsc.VectorSubcoreMesh(core_axis_name="c", subcore_axis_name="s")
    return pl.kernel(
        _sc_hist_body,
        out_type=jax.ShapeDtypeStruct((NCHUNK, TILE_WORDS), jnp.float32),
        mesh=mesh,
        compiler_params=pltpu.CompilerParams(needs_layout_passes=False),
        scratch_types=[
            pltpu.VMEM((BATCH_PTS,), jnp.int32),
            pltpu.VMEM((BATCH_PTS,), jnp.float32),
            pltpu.VMEM((TILE_WORDS,), jnp.float32),
        ],
    )(idx_flat, w_flat)


def _combine_body(tc_ref, tl_ref, tr_ref, out_ref):
    b = pl.program_id(0)
    tc = tc_ref[0]
    tl = tl_ref[0]
    tr = tr_ref[0]
    acc = tc[:, 1:CHUNK + 1]
    left = jnp.concatenate(
        [tl[:, CHUNK + 1:CHUNK + 5], jnp.zeros((NB_F, CHUNK - 4), jnp.float32)],
        axis=1)
    right = jnp.concatenate(
        [jnp.zeros((NB_F, CHUNK - 1), jnp.float32), tr[:, 0:1]], axis=1)
    acc = acc + jnp.where(b > 0, left, 0.0) + jnp.where(b < NCHUNK - 1, right, 0.0)
    out_ref[...] = 20.0 * jnp.log10(jnp.maximum(jnp.float32(1e-06), acc))


def _run_combine(tiles, nb_t):
    tiles3 = tiles.reshape(NCHUNK, NB_F, TILE_W)
    tile_spec_c = pl.BlockSpec((1, NB_F, TILE_W), lambda b: (b, 0, 0))
    tile_spec_l = pl.BlockSpec((1, NB_F, TILE_W),
                               lambda b: (jnp.maximum(b - 1, 0), 0, 0))
    tile_spec_r = pl.BlockSpec((1, NB_F, TILE_W),
                               lambda b: (jnp.minimum(b + 1, NCHUNK - 1), 0, 0))
    return pl.pallas_call(
        _combine_body,
        grid=(NCHUNK,),
        in_specs=[tile_spec_c, tile_spec_l, tile_spec_r],
        out_specs=pl.BlockSpec((NB_F, CHUNK), lambda b: (0, b)),
        out_shape=jax.ShapeDtypeStruct((NB_F, nb_t), jnp.float32),
    )(tiles3, tiles3, tiles3)


def kernel(signal, window):
    spec = _stft_frames_major(signal, window)
    spec_mag = jnp.abs(spec) / spec.shape[1]

    ts = jnp.roll(signal, 1).at[0].set(0.0)
    spec_ts = _stft_frames_major(ts, window)
    inst_freqs = _arg(spec * jnp.conj(spec_ts))

    fs = jnp.roll(spec, 1, axis=1).at[:, 0].set(0.0)
    time_delays = 0.5 - _arg(spec * jnp.conj(fs))

    win_duration = WIN_LENGTH / SR
    duration = signal.shape[0] / SR
    win_start_times = jnp.arange(0.0, duration, HOP / SR)
    eps = float(np.finfo(np.float32).eps)
    win_center = win_start_times + win_duration / 2 + eps
    reassigned_times = win_center[:, None] + time_delays * win_duration

    n_frames, nb_f = spec.shape
    output_frame_count = int(np.ceil(duration * SR / HOP))
    t_hi = output_frame_count * HOP / SR
    nb_t = output_frame_count
    wf = (0.5 - 0.0) / nb_f
    wt = (t_hi - 0.0) / nb_t

    padding = ((0, FPAD - n_frames), (0, KPAD - nb_f))
    f2 = jnp.pad(inst_freqs, padding)
    t2 = jnp.pad(reassigned_times, padding)
    w2 = jnp.pad(spec_mag, padding)

    idx_arr, wm = _run_prep(f2, t2, w2, nb_f, nb_t, wf, wt, t_hi)
    tiles = _run_sc_hist(idx_arr.reshape(-1), wm.reshape(-1))
    return _run_combine(tiles, nb_t)


# speedup vs baseline: 473.0953x; 459.4421x over previous
import functools
import math

import jax
import jax.numpy as jnp
import numpy as np
from jax import lax
from jax.experimental import pallas as pl
from jax.experimental.pallas import tpu as pltpu
from jax.experimental.pallas import tpu_sc as plsc

N_FFT = 1024
WIN_LENGTH = 1024
HOP = 256
SR = 44100

CHUNK = 128
NCHUNK = 65
FPAD = CHUNK * NCHUNK
KPAD = 528
TILE_W = 136
NB_F = 513
TILE_WORDS = NB_F * TILE_W
BATCH_FRAMES = 16
BATCH_PTS = BATCH_FRAMES * KPAD
NBATCH = CHUNK // BATCH_FRAMES


def _arg(values):
    return (jnp.angle(values) / (2.0 * np.pi)) % 1.0


FRAME_BLOCK = 128


def _framing_body(xp1_ref, xp2_ref, win_ref, out1_ref, out2_ref, *, n_frames):
    b = pl.program_id(0)
    win = win_ref[...]

    def row(i, carry):
        m = b * FRAME_BLOCK + i
        mc = jnp.minimum(m, n_frames - 1)
        off = mc * HOP
        seg1 = xp1_ref[pl.ds(off, N_FFT)]
        seg2 = xp2_ref[pl.ds(off, N_FFT)]
        out1_ref[pl.ds(i, 1), :] = (seg1 * win).reshape(1, N_FFT)
        out2_ref[pl.ds(i, 1), :] = (seg2 * win).reshape(1, N_FFT)
        return carry

    lax.fori_loop(0, FRAME_BLOCK, row, 0)


def _run_framing(xp1, xp2, window, n_frames):
    grid = (n_frames + FRAME_BLOCK - 1) // FRAME_BLOCK
    full1 = pl.BlockSpec(xp1.shape, lambda b: (0,))
    fullw = pl.BlockSpec(window.shape, lambda b: (0,))
    out_spec = pl.BlockSpec((FRAME_BLOCK, N_FFT), lambda b: (b, 0))
    return pl.pallas_call(
        functools.partial(_framing_body, n_frames=n_frames),
        grid=(grid,),
        in_specs=[full1, full1, fullw],
        out_specs=[out_spec, out_spec],
        out_shape=[
            jax.ShapeDtypeStruct((n_frames, N_FFT), jnp.float32),
            jax.ShapeDtypeStruct((n_frames, N_FFT), jnp.float32),
        ],
    )(xp1, xp2, window)


def _prep_body(f_ref, t_ref, w_ref, idx_ref, wm_ref, *, wf, wt, t_hi,
               nb_f, nb_t, block_rows):
    b = pl.program_id(0)
    f = f_ref[...]
    t = t_ref[...]
    w = w_ref[...]
    fi = jnp.clip(jnp.floor((f - 0.0) / wf).astype(jnp.int32), 0, nb_f - 1)
    ti = jnp.clip(jnp.floor(t / wt).astype(jnp.int32), 0, nb_t - 1)
    inb = (f >= 0.0) & (f <= 0.5) & (t >= 0.0) & (t <= t_hi)
    wm = jnp.where(inb, w, jnp.zeros_like(w))
    m = b * block_rows + lax.broadcasted_iota(jnp.int32, f.shape, 0)
    dt = jnp.clip(ti - m, -1, 4)
    mmod = m - (m // CHUNK) * CHUNK
    idx_ref[...] = fi * TILE_W + mmod + 1 + dt
    wm_ref[...] = wm


def _run_prep(f2, t2, w2, nb_f, nb_t, wf, wt, t_hi):
    block_rows = 128
    grid = FPAD // block_rows
    spec = pl.BlockSpec((block_rows, KPAD), lambda b: (b, 0))
    return pl.pallas_call(
        functools.partial(_prep_body, wf=wf, wt=wt, t_hi=t_hi, nb_f=nb_f,
                          nb_t=nb_t, block_rows=block_rows),
        grid=(grid,),
        in_specs=[spec, spec, spec],
        out_specs=[spec, spec],
        out_shape=[
            jax.ShapeDtypeStruct((FPAD, KPAD), jnp.int32),
            jax.ShapeDtypeStruct((FPAD, KPAD), jnp.float32),
        ],
    )(f2, t2, w2)


def _sc_hist_body(idx_hbm, w_hbm, tiles_hbm, idx_v, w_v, hist_v):
    cid = lax.axis_index("c")
    sid = lax.axis_index("s")
    wid = sid * 2 + cid

    def zero_body(i, carry):
        hist_v[pl.ds(i * 16, 16)] = jnp.zeros((16,), jnp.float32)
        return carry

    def scat_body(i, carry):
        iv = idx_v[pl.ds(i * 16, 16)]
        wv = w_v[pl.ds(i * 16, 16)]
        plsc.addupdate_scatter(hist_v, [iv], wv)
        return carry

    for rep in range(3):
        c = rep * 32 + wid

        @pl.when(c < NCHUNK)
        def _():
            lax.fori_loop(0, TILE_WORDS // 16, zero_body, 0)
            for j in range(NBATCH):
                off = pl.multiple_of(c * (CHUNK * KPAD) + j * BATCH_PTS, 8)
                pltpu.sync_copy(idx_hbm.at[pl.ds(off, BATCH_PTS)], idx_v)
                pltpu.sync_copy(w_hbm.at[pl.ds(off, BATCH_PTS)], w_v)
                lax.fori_loop(0, BATCH_PTS // 16, scat_body, 0)
            pltpu.sync_copy(hist_v, tiles_hbm.at[c])


def _run_sc_hist(idx_flat, w_flat):
    mesh = plsc.VectorSubcoreMesh(core_axis_name="c", subcore_axis_name="s")
    return pl.kernel(
        _sc_hist_body,
        out_type=jax.ShapeDtypeStruct((NCHUNK, TILE_WORDS), jnp.float32),
        mesh=mesh,
        compiler_params=pltpu.CompilerParams(needs_layout_passes=False),
        scratch_types=[
            pltpu.VMEM((BATCH_PTS,), jnp.int32),
            pltpu.VMEM((BATCH_PTS,), jnp.float32),
            pltpu.VMEM((TILE_WORDS,), jnp.float32),
        ],
    )(idx_flat, w_flat)


def _combine_body(tc_ref, tl_ref, tr_ref, out_ref):
    b = pl.program_id(0)
    tc = tc_ref[0]
    tl = tl_ref[0]
    tr = tr_ref[0]
    acc = tc[:, 1:CHUNK + 1]
    left = jnp.concatenate(
        [tl[:, CHUNK + 1:CHUNK + 5], jnp.zeros((NB_F, CHUNK - 4), jnp.float32)],
        axis=1)
    right = jnp.concatenate(
        [jnp.zeros((NB_F, CHUNK - 1), jnp.float32), tr[:, 0:1]], axis=1)
    acc = acc + jnp.where(b > 0, left, 0.0) + jnp.where(b < NCHUNK - 1, right, 0.0)
    out_ref[...] = 20.0 * jnp.log10(jnp.maximum(jnp.float32(1e-06), acc))


def _run_combine(tiles, nb_t):
    tiles3 = tiles.reshape(NCHUNK, NB_F, TILE_W)
    tile_spec_c = pl.BlockSpec((1, NB_F, TILE_W), lambda b: (b, 0, 0))
    tile_spec_l = pl.BlockSpec((1, NB_F, TILE_W),
                               lambda b: (jnp.maximum(b - 1, 0), 0, 0))
    tile_spec_r = pl.BlockSpec((1, NB_F, TILE_W),
                               lambda b: (jnp.minimum(b + 1, NCHUNK - 1), 0, 0))
    return pl.pallas_call(
        _combine_body,
        grid=(NCHUNK,),
        in_specs=[tile_spec_c, tile_spec_l, tile_spec_r],
        out_specs=pl.BlockSpec((NB_F, CHUNK), lambda b: (0, b)),
        out_shape=jax.ShapeDtypeStruct((NB_F, nb_t), jnp.float32),
    )(tiles3, tiles3, tiles3)


def kernel(signal, window):
    pad = N_FFT // 2
    n_frames = 1 + signal.shape[0] // HOP
    ts = jnp.roll(signal, 1).at[0].set(0.0)
    xp1 = jnp.pad(signal, (pad, pad), mode='reflect')
    xp2 = jnp.pad(ts, (pad, pad), mode='reflect')
    frames1, frames2 = _run_framing(xp1, xp2, window, n_frames)
    spec = jnp.fft.rfft(frames1, axis=1)
    spec_ts = jnp.fft.rfft(frames2, axis=1)
    spec_mag = jnp.abs(spec) / spec.shape[1]
    inst_freqs = _arg(spec * jnp.conj(spec_ts))

    fs = jnp.roll(spec, 1, axis=1).at[:, 0].set(0.0)
    time_delays = 0.5 - _arg(spec * jnp.conj(fs))

    win_duration = WIN_LENGTH / SR
    duration = signal.shape[0] / SR
    win_start_times = jnp.arange(0.0, duration, HOP / SR)
    eps = float(np.finfo(np.float32).eps)
    win_center = win_start_times + win_duration / 2 + eps
    reassigned_times = win_center[:, None] + time_delays * win_duration

    n_frames, nb_f = spec.shape
    output_frame_count = int(np.ceil(duration * SR / HOP))
    t_hi = output_frame_count * HOP / SR
    nb_t = output_frame_count
    wf = (0.5 - 0.0) / nb_f
    wt = (t_hi - 0.0) / nb_t

    padding = ((0, FPAD - n_frames), (0, KPAD - nb_f))
    f2 = jnp.pad(inst_freqs, padding)
    t2 = jnp.pad(reassigned_times, padding)
    w2 = jnp.pad(spec_mag, padding)

    idx_arr, wm = _run_prep(f2, t2, w2, nb_f, nb_t, wf, wt, t_hi)
    tiles = _run_sc_hist(idx_arr.reshape(-1), wm.reshape(-1))
    return _run_combine(tiles, nb_t)
